# baseline (speedup 1.0000x reference)
import functools

import numpy as np
import jax
import jax.numpy as jnp
from jax import lax
from jax.experimental import pallas as pl
from jax.experimental.pallas import tpu as pltpu
from jax.experimental.pallas import tpu_sc as plsc

Nx = 512
Ny = 512
Nu = 768
NTHETA = 90
NC = 2
NS = 16
L = 16
VPC = NTHETA // NC
RPT = Ny // NS
XB = Nx // L
ACC_N = VPC * Nu
SLICE = ACC_N // NS
TABN = 2048
TAB3 = 3 * TABN


def _make_tables():
    th = np.arange(NTHETA, dtype=np.float32) * np.float32(np.pi / NTHETA)
    cos_t = np.cos(th)
    sin_t = np.sin(th)
    ac, asn = np.abs(cos_t), np.abs(sin_t)
    h = np.minimum(1.0 / np.maximum(ac, 1e-12),
                   1.0 / np.maximum(asn, 1e-12)).astype(np.float32)
    b1 = np.abs(asn - ac)
    b2 = asn + ac
    r1 = (b2 - b1) * np.float32(0.5)
    r2 = (b2 + b1) * np.float32(0.5)
    r3 = b2
    bigA = h / (2.0 * np.maximum(r1, np.float32(1e-6)))
    atot = h * (b1 + b2) * np.float32(0.5)
    c0 = (-(Nx - 1) / 2.0) * cos_t - b2 * np.float32(0.5) \
        + np.float32((Nu - 1) / 2.0 + 256.0)
    par = np.stack([cos_t, sin_t, c0], axis=1)
    par = np.repeat(par.astype(np.float32)[:, :, None], L, axis=2)
    ytab = np.repeat(((np.arange(Ny, dtype=np.float32) - (Ny - 1) / 2.0)
                      )[:, None], L, axis=1)

    f = ((np.arange(TABN, dtype=np.float64) + 0.5) / TABN)[None, :]
    r1d, r2d, r3d = (x.astype(np.float64)[:, None] for x in (r1, r2, r3))
    hd = h.astype(np.float64)[:, None]
    Ad = bigA.astype(np.float64)[:, None]
    atd = atot.astype(np.float64)[:, None]

    def F(t):
        c1 = np.clip(t, 0.0, r1d)
        c2 = np.clip(t, r1d, r2d) - r1d
        c3 = np.clip(t, r2d, r3d) - r2d
        return Ad * (c1 - c3) * (c1 + c3) + hd * (c2 + c3)

    F1 = F(0.5 - f)
    F2 = F(1.5 - f)
    wtab = np.stack([F1, F2 - F1, atd - F2], axis=1)
    wtab = wtab.astype(np.float32).reshape(NTHETA * TAB3)
    return par, ytab, wtab


_PAR, _YTAB, _WTAB = _make_tables()

_mesh = plsc.VectorSubcoreMesh(core_axis_name="c", subcore_axis_name="s")


@functools.partial(
    pl.kernel,
    out_type=jax.ShapeDtypeStruct((NTHETA * Nu,), jnp.float32),
    mesh=_mesh,
    scratch_types=[
        pltpu.VMEM((RPT, Nx), jnp.float32),
        pltpu.VMEM((VPC, 3, L), jnp.float32),
        pltpu.VMEM((RPT, L), jnp.float32),
        pltpu.VMEM((2, TAB3), jnp.float32),
        pltpu.VMEM((NS * Nu,), jnp.float32),
        pltpu.VMEM((ACC_N,), jnp.float32),
        pltpu.VMEM((SLICE,), jnp.float32),
        pltpu.VMEM((SLICE,), jnp.float32),
        pltpu.VMEM_SHARED((NS, ACC_N), jnp.float32),
        pltpu.SemaphoreType.DMA,
    ],
    compiler_params=pltpu.CompilerParams(use_tc_tiling_on_sc=False,
                                         needs_layout_passes=False),
)
def _ct_project_sc(img_h, par_h, ytab_h, wtab_h, out_h,
                   img_v, par_v, ytab_v, tab_v, acc16, accv,
                   red_a, red_t, slots, sem):
    c = lax.axis_index("c")
    s = lax.axis_index("s")
    pltpu.sync_copy(img_h.at[pl.ds(s * RPT, RPT)], img_v)
    pltpu.sync_copy(par_h.at[pl.ds(c * VPC, VPC)], par_v)
    pltpu.sync_copy(ytab_h.at[pl.ds(s * RPT, RPT)], ytab_v)
    pltpu.sync_copy(wtab_h.at[pl.ds(c * VPC * TAB3, TAB3)], tab_v.at[0])

    zero = jnp.zeros((L,), jnp.float32)
    lane_off0 = jnp.arange(L, dtype=jnp.int32) * Nu - 256
    lane_off1 = lane_off0 + 1
    lane_off2 = lane_off0 + 2
    lane_f = jnp.arange(L, dtype=jnp.int32).astype(jnp.float32)
    cap = jnp.full((L,), TABN - 1, jnp.int32)
    off1 = jnp.full((L,), TABN, jnp.int32)
    off2 = jnp.full((L,), 2 * TABN, jnp.int32)

    @pl.loop(0, NS * Nu // L, unroll=8)
    def _zero(i):
        acc16[pl.ds(i * L, L)] = zero

    @pl.loop(0, VPC)
    def _view(v):
        p = lax.rem(v, 2)
        alpha = par_v[v, 0, :]
        beta = par_v[v, 1, :]
        c0 = par_v[v, 2, :]
        step16 = alpha * np.float32(L)

        @pl.when(v + 1 < VPC)
        def _prefetch():
            pltpu.async_copy(
                wtab_h.at[pl.ds((c * VPC + v + 1) * TAB3, TAB3)],
                tab_v.at[1 - p], sem)

        tab = tab_v.at[p]

        @pl.loop(0, RPT)
        def _row(r):
            cr = c0 + beta * ytab_v[r, :]
            u1i0 = alpha * lane_f + cr

            @pl.loop(0, XB, init_carry=u1i0, unroll=8)
            def _xblk(xb, u1i):
                bi = u1i.astype(jnp.int32)
                bf = bi.astype(jnp.float32)
                f = u1i - bf
                fi = jnp.minimum((f * np.float32(TABN)).astype(jnp.int32), cap)
                w0t = plsc.load_gather(tab, [fi])
                w1t = plsc.load_gather(tab, [fi + off1])
                w2t = plsc.load_gather(tab, [fi + off2])
                g = img_v[r, pl.ds(xb * L, L)]
                plsc.addupdate_scatter(acc16, [bi + lane_off0], w0t * g)
                plsc.addupdate_scatter(acc16, [bi + lane_off1], w1t * g)
                plsc.addupdate_scatter(acc16, [bi + lane_off2], w2t * g)
                return u1i + step16

        @pl.loop(0, Nu // L, unroll=2)
        def _reduce(j):
            t = acc16[pl.ds(j * L, L)]
            acc16[pl.ds(j * L, L)] = zero
            for lane in range(1, NS):
                t = t + acc16[pl.ds(lane * Nu + j * L, L)]
                acc16[pl.ds(lane * Nu + j * L, L)] = zero
            accv[pl.ds(v * Nu + j * L, L)] = t

        @pl.when(v + 1 < VPC)
        def _wait():
            pltpu.make_async_copy(
                wtab_h.at[pl.ds(0, TAB3)], tab_v.at[1 - p], sem).wait()

    pltpu.sync_copy(accv, slots.at[s])
    plsc.subcore_barrier()
    pltpu.sync_copy(slots.at[0, pl.ds(s * SLICE, SLICE)], red_a)

    @pl.loop(1, NS)
    def _slot(k):
        pltpu.sync_copy(slots.at[k, pl.ds(s * SLICE, SLICE)], red_t)

        @pl.loop(0, SLICE // L, unroll=8)
        def _add(i):
            red_a[pl.ds(i * L, L)] = red_a[pl.ds(i * L, L)] \
                + red_t[pl.ds(i * L, L)]

    pltpu.sync_copy(red_a, out_h.at[pl.ds(c * ACC_N + s * SLICE, SLICE)])


def kernel(img):
    out = _ct_project_sc(img, jnp.asarray(_PAR), jnp.asarray(_YTAB),
                         jnp.asarray(_WTAB))
    return out.reshape(NTHETA, Nu)


# speedup vs baseline: 1.2867x; 1.2867x over previous
import functools

import numpy as np
import jax
import jax.numpy as jnp
from jax import lax
from jax.experimental import pallas as pl
from jax.experimental.pallas import tpu as pltpu
from jax.experimental.pallas import tpu_sc as plsc

Nx = 512
Ny = 512
Nu = 768
NTHETA = 90
NC = 2
NS = 16
L = 16
VPC = NTHETA // NC
RPT = Ny // NS
XB = Nx // L
ACC_N = VPC * Nu
SLICE = ACC_N // NS
TABN = 2048
TAB3 = 3 * TABN


def _make_tables():
    th = np.arange(NTHETA, dtype=np.float32) * np.float32(np.pi / NTHETA)
    cos_t = np.cos(th)
    sin_t = np.sin(th)
    ac, asn = np.abs(cos_t), np.abs(sin_t)
    h = np.minimum(1.0 / np.maximum(ac, 1e-12),
                   1.0 / np.maximum(asn, 1e-12)).astype(np.float32)
    b1 = np.abs(asn - ac)
    b2 = asn + ac
    r1 = (b2 - b1) * np.float32(0.5)
    r2 = (b2 + b1) * np.float32(0.5)
    r3 = b2
    bigA = h / (2.0 * np.maximum(r1, np.float32(1e-6)))
    atot = h * (b1 + b2) * np.float32(0.5)
    c0 = (-(Nx - 1) / 2.0) * cos_t - b2 * np.float32(0.5) \
        + np.float32((Nu - 1) / 2.0 + 256.0)
    par = np.stack([cos_t * TABN, sin_t * TABN, c0 * TABN], axis=1)
    par = np.repeat(par.astype(np.float32)[:, :, None], L, axis=2)
    ytab = np.repeat(((np.arange(Ny, dtype=np.float32) - (Ny - 1) / 2.0)
                      )[:, None], L, axis=1)

    f = ((np.arange(TABN, dtype=np.float64) + 0.5) / TABN)[None, :]
    r1d, r2d, r3d = (x.astype(np.float64)[:, None] for x in (r1, r2, r3))
    hd = h.astype(np.float64)[:, None]
    Ad = bigA.astype(np.float64)[:, None]
    atd = atot.astype(np.float64)[:, None]

    def F(t):
        c1 = np.clip(t, 0.0, r1d)
        c2 = np.clip(t, r1d, r2d) - r1d
        c3 = np.clip(t, r2d, r3d) - r2d
        return Ad * (c1 - c3) * (c1 + c3) + hd * (c2 + c3)

    F1 = F(0.5 - f)
    F2 = F(1.5 - f)
    wtab = np.stack([F1, F2 - F1, atd - F2], axis=1)
    wtab = wtab.astype(np.float32).reshape(NTHETA * TAB3)
    return par, ytab, wtab


_PAR, _YTAB, _WTAB = _make_tables()

_mesh = plsc.VectorSubcoreMesh(core_axis_name="c", subcore_axis_name="s")


@functools.partial(
    pl.kernel,
    out_type=jax.ShapeDtypeStruct((NTHETA * Nu,), jnp.float32),
    mesh=_mesh,
    scratch_types=[
        pltpu.VMEM((RPT, Nx), jnp.float32),
        pltpu.VMEM((VPC, 3, L), jnp.float32),
        pltpu.VMEM((RPT, L), jnp.float32),
        pltpu.VMEM((2, TAB3), jnp.float32),
        pltpu.VMEM((NS * Nu,), jnp.float32),
        pltpu.VMEM((ACC_N,), jnp.float32),
        pltpu.VMEM((SLICE,), jnp.float32),
        pltpu.VMEM((SLICE,), jnp.float32),
        pltpu.VMEM_SHARED((NS, ACC_N), jnp.float32),
        pltpu.SemaphoreType.DMA,
    ],
    compiler_params=pltpu.CompilerParams(use_tc_tiling_on_sc=False,
                                         needs_layout_passes=False),
)
def _ct_project_sc(img_h, par_h, ytab_h, wtab_h, out_h,
                   img_v, par_v, ytab_v, tab_v, acc16, accv,
                   red_a, red_t, slots, sem):
    c = lax.axis_index("c")
    s = lax.axis_index("s")
    pltpu.sync_copy(img_h.at[pl.ds(s * RPT, RPT)], img_v)
    pltpu.sync_copy(par_h.at[pl.ds(c * VPC, VPC)], par_v)
    pltpu.sync_copy(ytab_h.at[pl.ds(s * RPT, RPT)], ytab_v)
    pltpu.sync_copy(wtab_h.at[pl.ds(c * VPC * TAB3, TAB3)], tab_v.at[0])

    zero = jnp.zeros((L,), jnp.float32)
    lane_off0 = jnp.arange(L, dtype=jnp.int32) * Nu - 256
    lane_off1 = lane_off0 + 1
    lane_off2 = lane_off0 + 2
    lane_f = jnp.arange(L, dtype=jnp.int32).astype(jnp.float32)
    sh11 = jnp.full((L,), 11, jnp.int32)
    m2047 = jnp.full((L,), TABN - 1, jnp.int32)

    @pl.loop(0, NS * Nu // L, unroll=8)
    def _zero(i):
        acc16[pl.ds(i * L, L)] = zero

    @pl.loop(0, VPC)
    def _view(v):
        p = lax.rem(v, 2)
        alpha = par_v[v, 0, :]
        beta = par_v[v, 1, :]
        c0 = par_v[v, 2, :]
        step16 = alpha * np.float32(L)

        @pl.when(v + 1 < VPC)
        def _prefetch():
            pltpu.async_copy(
                wtab_h.at[pl.ds((c * VPC + v + 1) * TAB3, TAB3)],
                tab_v.at[1 - p], sem)

        tab0 = tab_v.at[p, pl.ds(0, TABN)]
        tab1 = tab_v.at[p, pl.ds(TABN, TABN)]
        tab2 = tab_v.at[p, pl.ds(2 * TABN, TABN)]

        @pl.loop(0, RPT)
        def _row(r):
            cr = c0 + beta * ytab_v[r, :]
            u1i0 = alpha * lane_f + cr

            @plsc.parallel_loop(0, XB, carry=u1i0, unroll=8)
            def _xblk(xb, u1i):
                ti = u1i.astype(jnp.int32)
                bi = jnp.right_shift(ti, sh11)
                fi = jnp.bitwise_and(ti, m2047)
                w0t = plsc.load_gather(tab0, [fi])
                w1t = plsc.load_gather(tab1, [fi])
                w2t = plsc.load_gather(tab2, [fi])
                g = img_v[r, pl.ds(xb * L, L)]
                plsc.addupdate_scatter(acc16, [bi + lane_off0], w0t * g)
                plsc.addupdate_scatter(acc16, [bi + lane_off1], w1t * g)
                plsc.addupdate_scatter(acc16, [bi + lane_off2], w2t * g)
                return u1i + step16

        @pl.loop(0, Nu // L, unroll=2)
        def _reduce(j):
            t = acc16[pl.ds(j * L, L)]
            acc16[pl.ds(j * L, L)] = zero
            for lane in range(1, NS):
                t = t + acc16[pl.ds(lane * Nu + j * L, L)]
                acc16[pl.ds(lane * Nu + j * L, L)] = zero
            accv[pl.ds(v * Nu + j * L, L)] = t

        @pl.when(v + 1 < VPC)
        def _wait():
            pltpu.make_async_copy(
                wtab_h.at[pl.ds(0, TAB3)], tab_v.at[1 - p], sem).wait()

    pltpu.sync_copy(accv, slots.at[s])
    plsc.subcore_barrier()
    pltpu.sync_copy(slots.at[0, pl.ds(s * SLICE, SLICE)], red_a)

    @pl.loop(1, NS)
    def _slot(k):
        pltpu.sync_copy(slots.at[k, pl.ds(s * SLICE, SLICE)], red_t)

        @pl.loop(0, SLICE // L, unroll=8)
        def _add(i):
            red_a[pl.ds(i * L, L)] = red_a[pl.ds(i * L, L)] \
                + red_t[pl.ds(i * L, L)]

    pltpu.sync_copy(red_a, out_h.at[pl.ds(c * ACC_N + s * SLICE, SLICE)])


def kernel(img):
    out = _ct_project_sc(img, jnp.asarray(_PAR), jnp.asarray(_YTAB),
                         jnp.asarray(_WTAB))
    return out.reshape(NTHETA, Nu)


# speedup vs baseline: 2.4191x; 1.8801x over previous
import functools

import numpy as np
import jax
import jax.numpy as jnp
from jax import lax
from jax.experimental import pallas as pl
from jax.experimental.pallas import tpu as pltpu
from jax.experimental.pallas import tpu_sc as plsc

Nx = 512
Ny = 512
Nu = 768
NTHETA = 90
NC = 2
NS = 16
L = 16
VPC = NTHETA // NC
RPT = Ny // NS
XB = Nx // L
ACC_N = VPC * Nu
SLICE = ACC_N // NS
TABN = 2048
TAB2 = 2 * TABN


def _make_tables():
    th = np.arange(NTHETA, dtype=np.float32) * np.float32(np.pi / NTHETA)
    cos_t = np.cos(th)
    sin_t = np.sin(th)
    ac, asn = np.abs(cos_t), np.abs(sin_t)
    h = np.minimum(1.0 / np.maximum(ac, 1e-12),
                   1.0 / np.maximum(asn, 1e-12)).astype(np.float32)
    b1 = np.abs(asn - ac)
    b2 = asn + ac
    r1 = (b2 - b1) * np.float32(0.5)
    r2 = (b2 + b1) * np.float32(0.5)
    r3 = b2
    bigA = h / (2.0 * np.maximum(r1, np.float32(1e-6)))
    atot = h * (b1 + b2) * np.float32(0.5)

    x_type = ac >= asn
    order = []
    nxv = []
    for cidx in range(NC):
        vs = np.arange(cidx * VPC, (cidx + 1) * VPC)
        xs = [v for v in vs if x_type[v]]
        ys = [v for v in vs if not x_type[v]]
        nxv.append(len(xs))
        order.extend(xs + ys)
    order = np.asarray(order, np.int32)
    inv = np.argsort(order).astype(np.int32)

    P = np.where(x_type, cos_t, sin_t)
    Q = np.where(x_type, sin_t, cos_t)
    R = -(Nx - 1) / 2.0 * (P + Q) - b2 * np.float32(0.5) \
        + np.float32((Nu - 1) / 2.0 + 256.0)
    par = np.stack([P * TABN, Q * TABN, R * TABN, atot],
                   axis=1).astype(np.float32)
    par = par[order]
    par = np.repeat(par[:, :, None], L, axis=2)

    ctab = np.repeat(np.arange(Ny, dtype=np.float32)[:, None], L, axis=1)
    voff = (np.arange(VPC, dtype=np.int32) * Nu - 256)[:, None] \
        + np.zeros((1, L), np.int32)

    f = ((np.arange(TABN, dtype=np.float64) + 0.5) / TABN)[None, :]
    r1d, r2d, r3d = (x.astype(np.float64)[:, None] for x in (r1, r2, r3))
    hd = h.astype(np.float64)[:, None]
    Ad = bigA.astype(np.float64)[:, None]
    atd = atot.astype(np.float64)[:, None]

    def F(t):
        c1 = np.clip(t, 0.0, r1d)
        c2 = np.clip(t, r1d, r2d) - r1d
        c3 = np.clip(t, r2d, r3d) - r2d
        return Ad * (c1 - c3) * (c1 + c3) + hd * (c2 + c3)

    F1 = F(0.5 - f)
    F2 = F(1.5 - f)
    wtab = np.stack([F1, atd - F2], axis=1)
    wtab = wtab.astype(np.float32)[order].reshape(NTHETA * TAB2)
    return par, ctab, voff, wtab, np.asarray(nxv, np.int32), inv


_PAR, _CTAB, _VOFF, _WTAB, _NXV, _INV = _make_tables()

_mesh = plsc.VectorSubcoreMesh(core_axis_name="c", subcore_axis_name="s")


@functools.partial(
    pl.kernel,
    out_type=jax.ShapeDtypeStruct((NTHETA * Nu,), jnp.float32),
    mesh=_mesh,
    scratch_types=[
        pltpu.VMEM((RPT, Nx), jnp.float32),
        pltpu.VMEM((RPT, Ny), jnp.float32),
        pltpu.VMEM((VPC, 4, L), jnp.float32),
        pltpu.VMEM((VPC, L), jnp.int32),
        pltpu.VMEM((RPT, L), jnp.float32),
        pltpu.VMEM((2, TAB2), jnp.float32),
        pltpu.VMEM((ACC_N,), jnp.float32),
        pltpu.VMEM((SLICE,), jnp.float32),
        pltpu.VMEM((SLICE,), jnp.float32),
        pltpu.VMEM_SHARED((NS, ACC_N), jnp.float32),
        pltpu.SemaphoreType.DMA,
    ],
    compiler_params=pltpu.CompilerParams(use_tc_tiling_on_sc=False,
                                         needs_layout_passes=False),
)
def _ct_project_sc(img_h, imgt_h, par_h, voff_h, ctab_h, wtab_h, out_h,
                   img_v, imgt_v, par_v, voff_v, ctab_v, tab_v, accv,
                   red_a, red_t, slots, sem):
    c = lax.axis_index("c")
    s = lax.axis_index("s")
    nxc = jnp.int32(_NXV[0]) + (jnp.int32(_NXV[1] - _NXV[0])) * c
    pltpu.sync_copy(img_h.at[pl.ds(s * RPT, RPT)], img_v)
    pltpu.sync_copy(imgt_h.at[pl.ds(s * RPT, RPT)], imgt_v)
    pltpu.sync_copy(par_h.at[pl.ds(c * VPC, VPC)], par_v)
    pltpu.sync_copy(voff_h, voff_v)
    pltpu.sync_copy(ctab_h.at[pl.ds(s * RPT, RPT)], ctab_v)
    pltpu.sync_copy(wtab_h.at[pl.ds(c * VPC * TAB2, TAB2)], tab_v.at[0])

    zero = jnp.zeros((L,), jnp.float32)
    lane_f = jnp.arange(L, dtype=jnp.int32).astype(jnp.float32)
    sh11 = jnp.full((L,), 11, jnp.int32)
    m2047 = jnp.full((L,), TABN - 1, jnp.int32)
    one_i = jnp.full((L,), 1, jnp.int32)
    two_i = jnp.full((L,), 2, jnp.int32)

    @pl.loop(0, ACC_N // L, unroll=8)
    def _zero(i):
        accv[pl.ds(i * L, L)] = zero

    @pl.loop(0, VPC)
    def _view(v):
        p = lax.rem(v, 2)
        coefP = par_v[v, 0, :]
        coefQ = par_v[v, 1, :]
        coefR = par_v[v, 2, :]
        atotv = par_v[v, 3, :]
        voffv = voff_v[v, :]
        step16 = coefP * np.float32(L)

        @pl.when(v + 1 < VPC)
        def _prefetch():
            pltpu.async_copy(
                wtab_h.at[pl.ds((c * VPC + v + 1) * TAB2, TAB2)],
                tab_v.at[1 - p], sem)

        tab0 = tab_v.at[p, pl.ds(0, TABN)]
        tab2 = tab_v.at[p, pl.ds(TABN, TABN)]

        def line_loop(gsrc):
            @pl.loop(0, RPT, step=4)
            def _line(r):
                cr = coefR + coefQ * ctab_v[r, :]
                u1i0a = coefP * lane_f + cr
                u1i0b = u1i0a + coefQ
                u1i0c = u1i0b + coefQ
                u1i0d = u1i0c + coefQ

                @plsc.parallel_loop(0, XB, carry=(u1i0a, u1i0b, u1i0c, u1i0d),
                                    unroll=4)
                def _blk(xb, u1i):
                    u1ia, u1ib, u1ic, u1id = u1i
                    for u1ix, rr in ((u1ia, r), (u1ib, r + 1),
                                     (u1ic, r + 2), (u1id, r + 3)):
                        ti = u1ix.astype(jnp.int32)
                        bi = jnp.right_shift(ti, sh11)
                        fi = jnp.bitwise_and(ti, m2047)
                        w0t = plsc.load_gather(tab0, [fi])
                        w2t = plsc.load_gather(tab2, [fi])
                        w1t = atotv - w0t - w2t
                        g = gsrc[rr, pl.ds(xb * L, L)]
                        i0 = bi + voffv
                        plsc.addupdate_scatter(accv, [i0], w0t * g)
                        plsc.addupdate_scatter(accv, [i0 + one_i], w1t * g)
                        plsc.addupdate_scatter(accv, [i0 + two_i], w2t * g)
                    return (u1ia + step16, u1ib + step16,
                            u1ic + step16, u1id + step16)

        @pl.when(v < nxc)
        def _xorient():
            line_loop(img_v)

        @pl.when(v >= nxc)
        def _yorient():
            line_loop(imgt_v)

        @pl.when(v + 1 < VPC)
        def _wait():
            pltpu.make_async_copy(
                wtab_h.at[pl.ds(0, TAB2)], tab_v.at[1 - p], sem).wait()

    pltpu.sync_copy(accv, slots.at[s])
    plsc.subcore_barrier()
    pltpu.sync_copy(slots.at[0, pl.ds(s * SLICE, SLICE)], red_a)

    @pl.loop(1, NS)
    def _slot(k):
        pltpu.sync_copy(slots.at[k, pl.ds(s * SLICE, SLICE)], red_t)

        @pl.loop(0, SLICE // L, unroll=8)
        def _add(i):
            red_a[pl.ds(i * L, L)] = red_a[pl.ds(i * L, L)] \
                + red_t[pl.ds(i * L, L)]

    pltpu.sync_copy(red_a, out_h.at[pl.ds(c * ACC_N + s * SLICE, SLICE)])


def kernel(img):
    out = _ct_project_sc(img, img.T, jnp.asarray(_PAR), jnp.asarray(_VOFF),
                         jnp.asarray(_CTAB), jnp.asarray(_WTAB))
    return out.reshape(NTHETA, Nu)[jnp.asarray(_INV)]
